# baseline (speedup 1.0000x reference)
import functools

import jax
import jax.numpy as jnp
from jax.experimental import pallas as pl
from jax.experimental.pallas import tpu as pltpu

T0 = 256
NS = 1
T1 = 4096


def _dot_t(a, b):
    return jax.lax.dot_general(
        a, b, (((1,), (1,)), ((), ())), preferred_element_type=jnp.float32)


def _body(*refs, tsteps, n, dh):
    ew_refs = refs[:NS]
    (x_ref, w0_ref, bn0_ref, b0_ref, w1_ref, bn1_ref, b1_ref,
     out_ref, avg_s, h_s, stats_s, w0f_s, w1f_s, bias_s) = refs[NS:]
    s = pl.program_id(0)
    nt0 = n // T0
    nt1 = n // T1
    p0 = tsteps * nt0
    p1 = tsteps * nt1
    mtok = tsteps * n

    @pl.when(s < p0)
    def _aggregate():
        @pl.when(s == 0)
        def _init():
            stats_s[...] = jnp.zeros_like(stats_s)

        t = s // nt0
        x = x_ref[0]
        xa = jnp.concatenate([x, jnp.ones_like(x)], axis=1)
        sub = T0 // NS
        asum = jnp.zeros((1, dh), jnp.float32)
        asq = jnp.zeros((1, dh), jnp.float32)
        for q, ref in enumerate(ew_refs):
            ew = ref[0]
            acc = jax.lax.dot_general(
                ew, xa, (((1,), (0,)), ((), ())),
                preferred_element_type=jnp.float32,
                precision=jax.lax.Precision.DEFAULT)
            z = acc[:, dh:dh + 1]
            z = jnp.where(z == 0.0, 1.0, z)
            avg = acc[:, :dh] / z
            avg_s[pl.ds(s * T0 + q * sub, sub), :] = avg
            asum += jnp.sum(avg, axis=0, keepdims=True)
            asq += jnp.sum(avg * avg, axis=0, keepdims=True)
        stats_s[0:1, :] += asum
        stats_s[1:2, :] += asq

        j = s % nt0
        xc = x_ref[0, pl.ds(j * T0, T0), :]
        stats_s[pl.ds(2 + t, 1), :] += jnp.sum(xc, axis=0, keepdims=True)
        stats_s[pl.ds(2 + tsteps + t, 1), :] += jnp.sum(xc * xc, axis=0,
                                                        keepdims=True)

    @pl.when(jnp.logical_and(s >= p0, s < p0 + p1))
    def _layer1():
        @pl.when(s == p0)
        def _fold0():
            node_sums = stats_s[2:2 + tsteps, :]
            node_sqs = stats_s[2 + tsteps:2 + 2 * tsteps, :]
            sum_x = jnp.sum(node_sums, axis=0, keepdims=True)
            sq_x = jnp.sum(node_sqs, axis=0, keepdims=True)
            sum_p = sum_x - node_sums[0:1, :]
            sq_p = sq_x - node_sqs[0:1, :]
            m0 = jnp.concatenate([sum_x, sum_p, stats_s[0:1, :]], 1) / mtok
            v0 = jnp.concatenate([sq_x, sq_p, stats_s[1:2, :]], 1) / mtok
            v0 = v0 - m0 * m0
            s0 = bn0_ref[0:1, :] * jax.lax.rsqrt(v0 + 1e-5)
            c0 = bn0_ref[1:2, :] - m0 * s0
            w0f_s[...] = w0_ref[...] * s0
            bias_s[0:1, :] = b0_ref[...] + _dot_t(c0, w0_ref[...])
            stats_s[pl.ds(2 + 2 * tsteps, 2), :] = jnp.zeros(
                (2, stats_s.shape[1]), jnp.float32)

        q = s - p0
        t = q // nt1
        j = q % nt1
        xt = x_ref[0, pl.ds(j * T1, T1), :]
        avg_t = avg_s[pl.ds(q * T1, T1), :]
        mask = (t > 0).astype(jnp.float32)
        wx = w0f_s[:, :dh] + mask * w0f_s[:, dh:2 * dh]
        h = _dot_t(xt, wx) + _dot_t(avg_t, w0f_s[:, 2 * dh:]) + bias_s[0:1, :]
        h = jnp.maximum(h, 0.0)
        h_s[pl.ds(q * T1, T1), :] = h
        stats_s[pl.ds(2 + 2 * tsteps, 1), :] += jnp.sum(h, 0, keepdims=True)
        stats_s[pl.ds(3 + 2 * tsteps, 1), :] += jnp.sum(h * h, 0,
                                                        keepdims=True)

    @pl.when(s >= p0 + p1)
    def _layer2():
        @pl.when(s == p0 + p1)
        def _fold1():
            m1 = stats_s[pl.ds(2 + 2 * tsteps, 1), :] / mtok
            v1 = stats_s[pl.ds(3 + 2 * tsteps, 1), :] / mtok - m1 * m1
            s1 = bn1_ref[0:1, :] * jax.lax.rsqrt(v1 + 1e-5)
            c1 = bn1_ref[1:2, :] - m1 * s1
            w1f_s[...] = w1_ref[...] * s1
            bias_s[1:2, :] = b1_ref[...] + _dot_t(c1, w1_ref[...])

        q = s - (p0 + p1)
        ht = h_s[pl.ds(q * T1, T1), :]
        out = _dot_t(ht, w1f_s[...]) + bias_s[1:2, :]
        out_ref[0] = jnp.maximum(out, 0.0)


@jax.jit
def kernel(node_data, edge_weights, W0, b0, W1, b1, bn0_g, bn0_b, bn1_g, bn1_b):
    t, n, dh = node_data.shape
    nt0 = n // T0
    nt1 = n // T1
    p0 = t * nt0
    p1 = t * nt1
    steps = p0 + 2 * p1

    bn0 = jnp.stack([bn0_g, bn0_b])
    bn1 = jnp.stack([bn1_g, bn1_b])

    def ew_idx(q):
        def idx(s):
            return (jnp.where(s < p0, s // nt0, t - 1),
                    jnp.where(s < p0, NS * (s % nt0) + q, NS * nt0 - NS + q),
                    0)
        return idx

    def x_idx(s):
        return (jnp.where(s < p0, s // nt0,
                          jnp.where(s < p0 + p1, (s - p0) // nt1, t - 1)),
                0, 0)

    def out_idx(s):
        q = s - (p0 + p1)
        return (jnp.where(s < p0 + p1, 0, q // nt1),
                jnp.where(s < p0 + p1, 0, q % nt1), 0)

    body = functools.partial(_body, tsteps=t, n=n, dh=dh)
    out = pl.pallas_call(
        body,
        grid=(steps,),
        in_specs=[
            *[pl.BlockSpec((1, T0 // NS, n), ew_idx(q)) for q in range(NS)],
            pl.BlockSpec((1, n, dh), x_idx),
            pl.BlockSpec((dh, 3 * dh), lambda s: (0, 0)),
            pl.BlockSpec((2, 3 * dh), lambda s: (0, 0)),
            pl.BlockSpec((1, dh), lambda s: (0, 0)),
            pl.BlockSpec((dh, dh), lambda s: (0, 0)),
            pl.BlockSpec((2, dh), lambda s: (0, 0)),
            pl.BlockSpec((1, dh), lambda s: (0, 0)),
        ],
        out_specs=pl.BlockSpec((1, T1, dh), out_idx),
        out_shape=jax.ShapeDtypeStruct((t, n, dh), jnp.float32),
        scratch_shapes=[
            pltpu.VMEM((t * n, dh), jnp.float32),
            pltpu.VMEM((t * n, dh), jnp.float32),
            pltpu.VMEM((4 + 2 * t, dh), jnp.float32),
            pltpu.VMEM((dh, 3 * dh), jnp.float32),
            pltpu.VMEM((dh, dh), jnp.float32),
            pltpu.VMEM((2, dh), jnp.float32),
        ],
    )(*([edge_weights] * NS), node_data, W0, bn0, b0.reshape(1, dh),
      W1, bn1, b1.reshape(1, dh))
    return out


# speedup vs baseline: 1.2053x; 1.2053x over previous
import functools

import jax
import jax.numpy as jnp
from jax.experimental import pallas as pl
from jax.experimental.pallas import tpu as pltpu

T0 = 512
NS = 1
T1 = 4096


def _dot_t(a, b):
    return jax.lax.dot_general(
        a, b, (((1,), (1,)), ((), ())), preferred_element_type=jnp.float32)


def _dot_t_fast(a, b):
    return jax.lax.dot_general(
        a, b, (((1,), (1,)), ((), ())), preferred_element_type=jnp.float32,
        precision=jax.lax.Precision.DEFAULT)


def _body(*refs, tsteps, n, dh):
    ew_refs = refs[:NS]
    (x_ref, w0_ref, bn0_ref, b0_ref, w1_ref, bn1_ref, b1_ref,
     out_ref, avg_s, h_s, stats_s, w0f_s, w1f_s, bias_s) = refs[NS:]
    s = pl.program_id(0)
    nt0 = n // T0
    nt1 = n // T1
    p0 = tsteps * nt0
    p1 = tsteps * nt1
    mtok = tsteps * n

    @pl.when(s < p0)
    def _aggregate():
        @pl.when(s == 0)
        def _init():
            stats_s[...] = jnp.zeros_like(stats_s)

        t = s // nt0
        x = x_ref[0]
        xa = jnp.concatenate([x, jnp.ones_like(x)], axis=1)
        sub = T0 // NS
        asum = jnp.zeros((1, dh), jnp.float32)
        asq = jnp.zeros((1, dh), jnp.float32)
        for q, ref in enumerate(ew_refs):
            ew = ref[0]
            acc = jax.lax.dot_general(
                ew, xa, (((1,), (0,)), ((), ())),
                preferred_element_type=jnp.float32,
                precision=jax.lax.Precision.DEFAULT)
            z = acc[:, dh:dh + 1]
            z = jnp.where(z == 0.0, 1.0, z)
            avg = acc[:, :dh] / z
            avg_s[pl.ds(s * T0 + q * sub, sub), :] = avg
            asum += jnp.sum(avg, axis=0, keepdims=True)
            asq += jnp.sum(avg * avg, axis=0, keepdims=True)
        stats_s[0:1, :] += asum
        stats_s[1:2, :] += asq

        j = s % nt0
        xc = x_ref[0, pl.ds(j * T0, T0), :]
        stats_s[pl.ds(2 + t, 1), :] += jnp.sum(xc, axis=0, keepdims=True)
        stats_s[pl.ds(2 + tsteps + t, 1), :] += jnp.sum(xc * xc, axis=0,
                                                        keepdims=True)

    @pl.when(jnp.logical_and(s >= p0, s < p0 + p1))
    def _layer1():
        @pl.when(s == p0)
        def _fold0():
            node_sums = stats_s[2:2 + tsteps, :]
            node_sqs = stats_s[2 + tsteps:2 + 2 * tsteps, :]
            sum_x = jnp.sum(node_sums, axis=0, keepdims=True)
            sq_x = jnp.sum(node_sqs, axis=0, keepdims=True)
            sum_p = sum_x - node_sums[0:1, :]
            sq_p = sq_x - node_sqs[0:1, :]
            m0 = jnp.concatenate([sum_x, sum_p, stats_s[0:1, :]], 1) / mtok
            v0 = jnp.concatenate([sq_x, sq_p, stats_s[1:2, :]], 1) / mtok
            v0 = v0 - m0 * m0
            s0 = bn0_ref[0:1, :] * jax.lax.rsqrt(v0 + 1e-5)
            c0 = bn0_ref[1:2, :] - m0 * s0
            w0f_s[...] = w0_ref[...] * s0
            bias_s[0:1, :] = b0_ref[...] + _dot_t(c0, w0_ref[...])
            stats_s[pl.ds(2 + 2 * tsteps, 2), :] = jnp.zeros(
                (2, stats_s.shape[1]), jnp.float32)

        q = s - p0
        t = q // nt1
        j = q % nt1
        xt = x_ref[0, pl.ds(j * T1, T1), :]
        avg_t = avg_s[pl.ds(q * T1, T1), :]
        mask = (t > 0).astype(jnp.float32)
        wx = w0f_s[:, :dh] + mask * w0f_s[:, dh:2 * dh]
        h = (_dot_t_fast(xt, wx) + _dot_t_fast(avg_t, w0f_s[:, 2 * dh:])
             + bias_s[0:1, :])
        h = jnp.maximum(h, 0.0)
        h_s[pl.ds(q * T1, T1), :] = h
        stats_s[pl.ds(2 + 2 * tsteps, 1), :] += jnp.sum(h, 0, keepdims=True)
        stats_s[pl.ds(3 + 2 * tsteps, 1), :] += jnp.sum(h * h, 0,
                                                        keepdims=True)

    @pl.when(s >= p0 + p1)
    def _layer2():
        @pl.when(s == p0 + p1)
        def _fold1():
            m1 = stats_s[pl.ds(2 + 2 * tsteps, 1), :] / mtok
            v1 = stats_s[pl.ds(3 + 2 * tsteps, 1), :] / mtok - m1 * m1
            s1 = bn1_ref[0:1, :] * jax.lax.rsqrt(v1 + 1e-5)
            c1 = bn1_ref[1:2, :] - m1 * s1
            w1f_s[...] = w1_ref[...] * s1
            bias_s[1:2, :] = b1_ref[...] + _dot_t(c1, w1_ref[...])

        q = s - (p0 + p1)
        ht = h_s[pl.ds(q * T1, T1), :]
        out = _dot_t_fast(ht, w1f_s[...]) + bias_s[1:2, :]
        out_ref[0] = jnp.maximum(out, 0.0)


@jax.jit
def kernel(node_data, edge_weights, W0, b0, W1, b1, bn0_g, bn0_b, bn1_g, bn1_b):
    t, n, dh = node_data.shape
    nt0 = n // T0
    nt1 = n // T1
    p0 = t * nt0
    p1 = t * nt1
    steps = p0 + 2 * p1

    bn0 = jnp.stack([bn0_g, bn0_b])
    bn1 = jnp.stack([bn1_g, bn1_b])

    def ew_idx(q):
        def idx(s):
            return (jnp.where(s < p0, s // nt0, t - 1),
                    jnp.where(s < p0, NS * (s % nt0) + q, NS * nt0 - NS + q),
                    0)
        return idx

    def x_idx(s):
        return (jnp.where(s < p0, s // nt0,
                          jnp.where(s < p0 + p1, (s - p0) // nt1, t - 1)),
                0, 0)

    def out_idx(s):
        q = s - (p0 + p1)
        return (jnp.where(s < p0 + p1, 0, q // nt1),
                jnp.where(s < p0 + p1, 0, q % nt1), 0)

    body = functools.partial(_body, tsteps=t, n=n, dh=dh)
    out = pl.pallas_call(
        body,
        grid=(steps,),
        in_specs=[
            *[pl.BlockSpec((1, T0 // NS, n), ew_idx(q)) for q in range(NS)],
            pl.BlockSpec((1, n, dh), x_idx),
            pl.BlockSpec((dh, 3 * dh), lambda s: (0, 0)),
            pl.BlockSpec((2, 3 * dh), lambda s: (0, 0)),
            pl.BlockSpec((1, dh), lambda s: (0, 0)),
            pl.BlockSpec((dh, dh), lambda s: (0, 0)),
            pl.BlockSpec((2, dh), lambda s: (0, 0)),
            pl.BlockSpec((1, dh), lambda s: (0, 0)),
        ],
        out_specs=pl.BlockSpec((1, T1, dh), out_idx),
        out_shape=jax.ShapeDtypeStruct((t, n, dh), jnp.float32),
        scratch_shapes=[
            pltpu.VMEM((t * n, dh), jnp.float32),
            pltpu.VMEM((t * n, dh), jnp.float32),
            pltpu.VMEM((4 + 2 * t, dh), jnp.float32),
            pltpu.VMEM((dh, 3 * dh), jnp.float32),
            pltpu.VMEM((dh, dh), jnp.float32),
            pltpu.VMEM((2, dh), jnp.float32),
        ],
    )(*([edge_weights] * NS), node_data, W0, bn0, b0.reshape(1, dh),
      W1, bn1, b1.reshape(1, dh))
    return out
